# baseline (speedup 1.0000x reference)
import functools

import jax
import jax.numpy as jnp
from jax import lax
from jax.experimental import pallas as pl
from jax.experimental.pallas import tpu as pltpu
from jax.experimental.pallas import tpu_sc as plsc

VOCAB = 1000000
DIM = 64
B = 16384
NEG = 20

NC = 2
NS = 16
NW = NC * NS
EX_PER_W = B // NW
E = 32
CHUNKS = EX_PER_W // E
NEG_E = E * NEG
NIDX_SLICES = NEG_E // 128


def _sc_scores(hidden_state, label_idxes, neg2d, out_table):
    mesh = plsc.VectorSubcoreMesh(core_axis_name="c", subcore_axis_name="s")

    @functools.partial(
        pl.kernel,
        mesh=mesh,
        compiler_params=pltpu.CompilerParams(
            needs_layout_passes=False, use_tc_tiling_on_sc=False),
        out_type=[
            jax.ShapeDtypeStruct((B,), jnp.float32),
            jax.ShapeDtypeStruct((B * NEG,), jnp.float32),
        ],
        scratch_types=[
            pltpu.VMEM((E,), jnp.int32),
            pltpu.VMEM((NEG_E,), jnp.int32),
            pltpu.VMEM((E, DIM), jnp.float32),
            pltpu.VMEM((E, DIM), jnp.float32),
            pltpu.VMEM((NEG_E, DIM), jnp.float32),
            pltpu.VMEM((E,), jnp.float32),
            pltpu.VMEM((NEG_E,), jnp.float32),
            pltpu.SemaphoreType.DMA,
        ],
    )
    def body(hid_hbm, lab_hbm, neg_hbm, tab_hbm, pos_out, neg_out,
             lidx, nidx, hrows, lrows, nrows, pos_s, neg_s, sem):
        wid = lax.axis_index("s") * NC + lax.axis_index("c")
        lanes = lax.iota(jnp.int32, 16)

        def chunk_body(k, carry):
            base_ex = wid * EX_PER_W + k * E

            pltpu.sync_copy(lab_hbm.at[pl.ds(base_ex, E)], lidx)
            pltpu.sync_copy(neg_hbm.at[pl.ds(base_ex * NEG, NEG_E)], nidx)
            pltpu.sync_copy(hid_hbm.at[pl.ds(base_ex, E)], hrows)

            cps = [pltpu.async_copy(tab_hbm.at[lidx], lrows, sem)]
            for j in range(NIDX_SLICES):
                cps.append(pltpu.async_copy(
                    tab_hbm.at[nidx.at[pl.ds(j * 128, 128)]],
                    nrows.at[pl.ds(j * 128, 128)], sem))
            for cp in cps:
                cp.wait()

            for g in range(E // 16):
                e_loc = g * 16 + lanes
                acc = jnp.zeros((16,), jnp.float32)
                for d in range(DIM):
                    dvec = jnp.full((16,), d, jnp.int32)
                    acc = acc + (plsc.load_gather(hrows, [e_loc, dvec]) *
                                 plsc.load_gather(lrows, [e_loc, dvec]))
                pos_s[pl.ds(g * 16, 16)] = acc

            def neg_group(g, carry2):
                c = g * 16 + lanes
                el = c // NEG
                acc = jnp.zeros((16,), jnp.float32)
                for d in range(DIM):
                    dvec = jnp.full((16,), d, jnp.int32)
                    acc = acc + (plsc.load_gather(nrows, [c, dvec]) *
                                 plsc.load_gather(hrows, [el, dvec]))
                neg_s[pl.ds(g * 16, 16)] = acc
                return carry2

            lax.fori_loop(0, NEG_E // 16, neg_group, jnp.int32(0))

            pltpu.sync_copy(pos_s, pos_out.at[pl.ds(base_ex, E)])
            pltpu.sync_copy(neg_s, neg_out.at[pl.ds(base_ex * NEG, NEG_E)])
            return carry

        lax.fori_loop(0, CHUNKS, chunk_body, jnp.int32(0))

    return body(hidden_state, label_idxes, neg2d, out_table)


def _tc_loss(pos2d, neg2d):
    def body(pos_ref, neg_ref, out_ref):
        p = pos_ref[...]
        n = neg_ref[...]
        pos_term = jnp.sum(jnp.minimum(p, 0.0) - jnp.log1p(jnp.exp(-jnp.abs(p))))
        neg_term = jnp.sum(jnp.minimum(-n, 0.0) - jnp.log1p(jnp.exp(-jnp.abs(n))))
        out_ref[...] = jnp.broadcast_to(-(pos_term + neg_term / NEG), (1, 1))

    return pl.pallas_call(
        body,
        out_shape=jax.ShapeDtypeStruct((1, 1), jnp.float32),
    )(pos2d, neg2d)


def kernel(hidden_state, label_idxes, neg_idxes, out_table):
    lab = label_idxes.astype(jnp.int32)
    neg = neg_idxes.astype(jnp.int32)
    pos_s, neg_s = _sc_scores(hidden_state, lab, neg, out_table)
    loss = _tc_loss(pos_s.reshape(B // 128, 128),
                    neg_s.reshape(B * NEG // 128, 128))
    return loss[0, 0]


# speedup vs baseline: 1.7160x; 1.7160x over previous
import functools

import jax
import jax.numpy as jnp
from jax import lax
from jax.experimental import pallas as pl
from jax.experimental.pallas import tpu as pltpu
from jax.experimental.pallas import tpu_sc as plsc

VOCAB = 1000000
DIM = 64
B = 16384
NEG = 20

NC = 2
NS = 16
NW = NC * NS
EX_PER_W = B // NW
E = 32
CHUNKS = EX_PER_W // E
ROWS_C = E * (1 + NEG)
GROUPS = ROWS_C // 16
NCHUNKS_TOT = B // E


def _sc_scores(hidden_state, idx_all, out_table):
    mesh = plsc.VectorSubcoreMesh(core_axis_name="c", subcore_axis_name="s")

    def row_gathers(tab_hbm, idx_buf, k, rows, hid, hid_hbm, sem):
        for j in range(5):
            pltpu.async_copy(
                tab_hbm.at[idx_buf.at[k, pl.ds(j * 128, 128)]],
                rows.at[pl.ds(j * 128, 128)], sem)
        pltpu.async_copy(
            tab_hbm.at[idx_buf.at[k, pl.ds(640, 32)]],
            rows.at[pl.ds(640, 32)], sem)
        pltpu.async_copy(hid_hbm.at[pl.ds(k * E, E)], hid, sem)

    def drain(tab_hbm, hid_hbm, rows, hid, sem):
        pltpu.make_async_copy(tab_hbm.at[pl.ds(0, ROWS_C)], rows, sem).wait()
        pltpu.make_async_copy(hid_hbm.at[pl.ds(0, E)], hid, sem).wait()

    @functools.partial(
        pl.kernel,
        mesh=mesh,
        compiler_params=pltpu.CompilerParams(
            needs_layout_passes=False, use_tc_tiling_on_sc=False),
        out_type=jax.ShapeDtypeStruct((NCHUNKS_TOT, ROWS_C), jnp.float32),
        scratch_types=[
            pltpu.VMEM((CHUNKS, ROWS_C), jnp.int32),
            pltpu.VMEM((ROWS_C, DIM), jnp.float32),
            pltpu.VMEM((ROWS_C, DIM), jnp.float32),
            pltpu.VMEM((E, DIM), jnp.float32),
            pltpu.VMEM((E, DIM), jnp.float32),
            pltpu.VMEM((ROWS_C,), jnp.float32),
            pltpu.VMEM((ROWS_C,), jnp.float32),
            pltpu.SemaphoreType.DMA,
            pltpu.SemaphoreType.DMA,
            pltpu.SemaphoreType.DMA,
            pltpu.SemaphoreType.DMA,
        ],
    )
    def body(hid_hbm, idx_hbm, tab_hbm, sc_out,
             idx_buf, rows0, rows1, hidb0, hidb1, sb0, sb1,
             sem0, sem1, ssem0, ssem1):
        wid = lax.axis_index("s") * NC + lax.axis_index("c")
        lanes = lax.iota(jnp.int32, 16)

        my_hid = hid_hbm.at[pl.ds(wid * EX_PER_W, EX_PER_W)]
        my_out = sc_out.at[pl.ds(wid * CHUNKS, CHUNKS)]

        pltpu.sync_copy(idx_hbm.at[pl.ds(wid * CHUNKS, CHUNKS)], idx_buf)

        def compute(rows, hid, scores):
            def group_body(g, carry):
                cand = g * 16 + lanes
                el = jnp.where(cand < E, cand, (cand - E) // NEG)
                acc = jnp.zeros((16,), jnp.float32)
                for d in range(DIM):
                    dvec = (lanes + d) & (DIM - 1)
                    acc = acc + (plsc.load_gather(rows, [cand, dvec]) *
                                 plsc.load_gather(hid, [el, dvec]))
                scores[pl.ds(g * 16, 16)] = acc
                return carry

            lax.fori_loop(0, GROUPS, group_body, jnp.int32(0))

        def halfstep(m, c, rows, hid, scores, sem, ssem, pf_c, pf_rows,
                     pf_hid, pf_sem):
            drain(tab_hbm, my_hid, rows, hid, sem)
            @pl.when(pf_c < CHUNKS)
            def _():
                row_gathers(tab_hbm, idx_buf, pf_c, pf_rows, pf_hid,
                            my_hid, pf_sem)
            @pl.when(m > 0)
            def _():
                _drain_scores(scores, ssem)
            compute(rows, hid, scores)
            pltpu.async_copy(scores, my_out.at[c], ssem)

        def _drain_scores(scores, ssem):
            pltpu.make_async_copy(my_out.at[0], scores, ssem).wait()

        row_gathers(tab_hbm, idx_buf, 0, rows0, hidb0, my_hid, sem0)

        def pair_body(m, carry):
            halfstep(m, 2 * m, rows0, hidb0, sb0, sem0, ssem0,
                     2 * m + 1, rows1, hidb1, sem1)
            halfstep(m, 2 * m + 1, rows1, hidb1, sb1, sem1, ssem1,
                     2 * m + 2, rows0, hidb0, sem0)
            return carry

        lax.fori_loop(0, CHUNKS // 2, pair_body, jnp.int32(0))

        _drain_scores(sb0, ssem0)
        _drain_scores(sb1, ssem1)

    return body(hidden_state, idx_all, out_table)


def _tc_loss(scores):
    def tc_body(s_ref, out_ref):
        x = s_ref[...]
        col = lax.broadcasted_iota(jnp.int32, x.shape, 1)
        is_pos = col < E
        m = jnp.where(is_pos, jnp.minimum(x, 0.0), -jnp.maximum(x, 0.0))
        t = m - jnp.log1p(jnp.exp(-jnp.abs(x)))
        w = jnp.where(is_pos, 1.0, 1.0 / NEG)
        out_ref[...] = jnp.broadcast_to(-jnp.sum(w * t), (1, 1))

    return pl.pallas_call(
        tc_body,
        out_shape=jax.ShapeDtypeStruct((1, 1), jnp.float32),
    )(scores)


def kernel(hidden_state, label_idxes, neg_idxes, out_table):
    lab = label_idxes.astype(jnp.int32).reshape(NCHUNKS_TOT, E)
    neg = neg_idxes.astype(jnp.int32).reshape(NCHUNKS_TOT, E * NEG)
    idx_all = jnp.concatenate([lab, neg], axis=1)
    scores = _sc_scores(hidden_state, idx_all, out_table)
    return _tc_loss(scores)[0, 0]
